# baseline (speedup 1.0000x reference)
import functools

import jax
import jax.numpy as jnp
from jax import lax
from jax.experimental import pallas as pl
from jax.experimental.pallas import tpu as pltpu
from jax.experimental.pallas import tpu_sc as plsc

MAX_TILES = 4
BATCH = 8
PATCHES = 1025
HIDDEN = 1280
LANES = 16
NCORES = 2
NSUB = 16
NWORKERS = NCORES * NSUB
ZERO_ROW = MAX_TILES * MAX_TILES

CHUNK = 128
CPS = 8
NCH = NWORKERS * CPS
TAIL_P = CPS * CHUNK
K = 8


def _sc_rows_body(eidx_ref, tab_ref, out_ref, idxbuf, e_buf, sem):
    c_idx = lax.axis_index("c")
    s_idx = lax.axis_index("s")
    w = s_idx * NCORES + c_idx

    pltpu.sync_copy(eidx_ref.at[w], idxbuf)
    pltpu.async_copy(tab_ref.at[idxbuf], e_buf, sem).wait()
    pltpu.sync_copy(e_buf, out_ref.at[w])


@jax.jit
def _sc_rows(eidx, table):
    mesh = plsc.VectorSubcoreMesh(core_axis_name="c", subcore_axis_name="s")
    k = functools.partial(
        pl.kernel,
        mesh=mesh,
        out_type=jax.ShapeDtypeStruct((NWORKERS, 8, HIDDEN), jnp.float32),
        scratch_types=[
            pltpu.VMEM((8,), jnp.int32),
            pltpu.VMEM((8, HIDDEN), jnp.float32),
            pltpu.SemaphoreType.DMA,
        ],
    )(_sc_rows_body)
    return k(eidx, table)


def _tc_body(hid_ref, rows_ref, out_ref, ibuf, obuf, tibuf, tobuf,
             sin, sout, tsin, tsout):

    def hid_slice(g):
        w = g // CPS
        ci = g % CPS
        return (w // MAX_TILES, w % MAX_TILES, pl.ds(ci * CHUNK, CHUNK))

    def copy_in(g, s):
        b, t, dp = hid_slice(g)
        return pltpu.make_async_copy(hid_ref.at[b, t, dp, :], ibuf.at[s],
                                     sin.at[s])

    def copy_out(g, s):
        b, t, dp = hid_slice(g)
        return pltpu.make_async_copy(obuf.at[s], out_ref.at[b, t, dp, :],
                                     sout.at[s])

    tail_in = pltpu.make_async_copy(
        hid_ref.at[:, :, pl.ds(TAIL_P, 1), :], tibuf, tsin)
    tail_in.start()

    for s in range(K):
        copy_in(s, s).start()

    def step(g, s):
        w = g // CPS
        copy_in(g, s).wait()
        r = rows_ref[pl.ds(w, 1), pl.ds(0, 1), :]

        @pl.when(g >= K)
        def _():
            copy_out(g - K, s).wait()

        obuf[s] = ibuf[s] + r[0]
        copy_out(g, s).start()

        @pl.when(g + K < NCH)
        def _():
            copy_in(g + K, s).start()

    def round_body(m, carry):
        g0 = m * K
        for s in range(K):
            step(g0 + s, s)
        return carry

    lax.fori_loop(0, NCH // K, round_body, 0)

    tail_in.wait()
    r_all = rows_ref[:, pl.ds(0, 1), :]
    tobuf[...] = tibuf[...] + r_all.reshape(BATCH, MAX_TILES, 1, HIDDEN)
    tail_out = pltpu.make_async_copy(
        tobuf, out_ref.at[:, :, pl.ds(TAIL_P, 1), :], tsout)
    tail_out.start()

    for s in range(K):
        copy_out(NCH - K + s, s).wait()
    tail_out.wait()


@jax.jit
def _tc_add(hidden_state, rows):
    return pl.pallas_call(
        _tc_body,
        in_specs=[
            pl.BlockSpec(memory_space=pl.ANY),
            pl.BlockSpec(memory_space=pltpu.VMEM),
        ],
        out_specs=pl.BlockSpec(memory_space=pl.ANY),
        out_shape=jax.ShapeDtypeStruct((BATCH, MAX_TILES, PATCHES, HIDDEN),
                                       jnp.float32),
        scratch_shapes=[
            pltpu.VMEM((K, CHUNK, HIDDEN), jnp.float32),
            pltpu.VMEM((K, CHUNK, HIDDEN), jnp.float32),
            pltpu.VMEM((BATCH, MAX_TILES, 1, HIDDEN), jnp.float32),
            pltpu.VMEM((BATCH, MAX_TILES, 1, HIDDEN), jnp.float32),
            pltpu.SemaphoreType.DMA((K,)),
            pltpu.SemaphoreType.DMA((K,)),
            pltpu.SemaphoreType.DMA,
            pltpu.SemaphoreType.DMA,
        ],
    )(hidden_state, rows)


def kernel(hidden_state, aspect_ratios, embedding, gate):
    scale = jnp.tanh(gate)[0]
    table = embedding.astype(jnp.float32).reshape(MAX_TILES * MAX_TILES,
                                                  HIDDEN) * scale
    table = jnp.concatenate([table, jnp.zeros((1, HIDDEN), jnp.float32)], axis=0)

    h = aspect_ratios[:, 0]
    wd = aspect_ratios[:, 1]
    n = h * wd
    p = jnp.arange(MAX_TILES, dtype=jnp.int32)
    sw = jnp.maximum(wd, 1)
    row = p[None, :] // sw[:, None]
    col = p[None, :] % sw[:, None]
    eidx = jnp.where(p[None, :] < n[:, None], row * MAX_TILES + col, ZERO_ROW)
    eidx = jnp.broadcast_to(eidx.reshape(NWORKERS, 1), (NWORKERS, 8))
    eidx = eidx.astype(jnp.int32)

    rows = _sc_rows(eidx, table)
    return _tc_add(hidden_state, rows)


# speedup vs baseline: 1.0010x; 1.0010x over previous
import functools

import jax
import jax.numpy as jnp
from jax import lax
from jax.experimental import pallas as pl
from jax.experimental.pallas import tpu as pltpu
from jax.experimental.pallas import tpu_sc as plsc

MAX_TILES = 4
BATCH = 8
PATCHES = 1025
HIDDEN = 1280
LANES = 16
NCORES = 2
NSUB = 16
NWORKERS = NCORES * NSUB
ZERO_ROW = MAX_TILES * MAX_TILES

CHUNK = 256
CPS = 4
NCH = NWORKERS * CPS
TAIL_P = CPS * CHUNK
K = 8


def _sc_rows_body(eidx_ref, tab_ref, out_ref, idxbuf, e_buf, sem):
    c_idx = lax.axis_index("c")
    s_idx = lax.axis_index("s")
    w = s_idx * NCORES + c_idx

    pltpu.sync_copy(eidx_ref.at[w], idxbuf)
    pltpu.async_copy(tab_ref.at[idxbuf], e_buf, sem).wait()
    pltpu.sync_copy(e_buf, out_ref.at[w])


@jax.jit
def _sc_rows(eidx, table):
    mesh = plsc.VectorSubcoreMesh(core_axis_name="c", subcore_axis_name="s")
    k = functools.partial(
        pl.kernel,
        mesh=mesh,
        out_type=jax.ShapeDtypeStruct((NWORKERS, 8, HIDDEN), jnp.float32),
        scratch_types=[
            pltpu.VMEM((8,), jnp.int32),
            pltpu.VMEM((8, HIDDEN), jnp.float32),
            pltpu.SemaphoreType.DMA,
        ],
    )(_sc_rows_body)
    return k(eidx, table)


def _tc_body(hid_ref, rows_ref, out_ref, ibuf, obuf, tibuf, tobuf,
             sin, sout, tsin, tsout):

    def hid_slice(g):
        w = g // CPS
        ci = g % CPS
        return (w // MAX_TILES, w % MAX_TILES, pl.ds(ci * CHUNK, CHUNK))

    def copy_in(g, s):
        b, t, dp = hid_slice(g)
        return pltpu.make_async_copy(hid_ref.at[b, t, dp, :], ibuf.at[s],
                                     sin.at[s])

    def copy_out(g, s):
        b, t, dp = hid_slice(g)
        return pltpu.make_async_copy(obuf.at[s], out_ref.at[b, t, dp, :],
                                     sout.at[s])

    tail_in = pltpu.make_async_copy(
        hid_ref.at[:, :, pl.ds(TAIL_P, 1), :], tibuf, tsin)
    tail_in.start()

    for s in range(K):
        copy_in(s, s).start()

    def step(g, s):
        w = g // CPS
        copy_in(g, s).wait()
        r = rows_ref[pl.ds(w, 1), pl.ds(0, 1), :]

        @pl.when(g >= K)
        def _():
            copy_out(g - K, s).wait()

        obuf[s] = ibuf[s] + r[0]
        copy_out(g, s).start()

        @pl.when(g + K < NCH)
        def _():
            copy_in(g + K, s).start()

    def round_body(m, carry):
        g0 = m * K
        for s in range(K):
            step(g0 + s, s)
        return carry

    lax.fori_loop(0, NCH // K, round_body, 0)

    tail_in.wait()
    r_all = rows_ref[:, pl.ds(0, 1), :]
    tobuf[...] = tibuf[...] + r_all.reshape(BATCH, MAX_TILES, 1, HIDDEN)
    tail_out = pltpu.make_async_copy(
        tobuf, out_ref.at[:, :, pl.ds(TAIL_P, 1), :], tsout)
    tail_out.start()

    for s in range(K):
        copy_out(NCH - K + s, s).wait()
    tail_out.wait()


@jax.jit
def _tc_add(hidden_state, rows):
    return pl.pallas_call(
        _tc_body,
        in_specs=[
            pl.BlockSpec(memory_space=pl.ANY),
            pl.BlockSpec(memory_space=pltpu.VMEM),
        ],
        out_specs=pl.BlockSpec(memory_space=pl.ANY),
        out_shape=jax.ShapeDtypeStruct((BATCH, MAX_TILES, PATCHES, HIDDEN),
                                       jnp.float32),
        scratch_shapes=[
            pltpu.VMEM((K, CHUNK, HIDDEN), jnp.float32),
            pltpu.VMEM((K, CHUNK, HIDDEN), jnp.float32),
            pltpu.VMEM((BATCH, MAX_TILES, 1, HIDDEN), jnp.float32),
            pltpu.VMEM((BATCH, MAX_TILES, 1, HIDDEN), jnp.float32),
            pltpu.SemaphoreType.DMA((K,)),
            pltpu.SemaphoreType.DMA((K,)),
            pltpu.SemaphoreType.DMA,
            pltpu.SemaphoreType.DMA,
        ],
    )(hidden_state, rows)


def kernel(hidden_state, aspect_ratios, embedding, gate):
    scale = jnp.tanh(gate)[0]
    table = embedding.astype(jnp.float32).reshape(MAX_TILES * MAX_TILES,
                                                  HIDDEN) * scale
    table = jnp.concatenate([table, jnp.zeros((1, HIDDEN), jnp.float32)], axis=0)

    h = aspect_ratios[:, 0]
    wd = aspect_ratios[:, 1]
    n = h * wd
    p = jnp.arange(MAX_TILES, dtype=jnp.int32)
    sw = jnp.maximum(wd, 1)
    row = p[None, :] // sw[:, None]
    col = p[None, :] % sw[:, None]
    eidx = jnp.where(p[None, :] < n[:, None], row * MAX_TILES + col, ZERO_ROW)
    eidx = jnp.broadcast_to(eidx.reshape(NWORKERS, 1), (NWORKERS, 8))
    eidx = eidx.astype(jnp.int32)

    rows = _sc_rows(eidx, table)
    return _tc_add(hidden_state, rows)
